# baseline (speedup 1.0000x reference)
import functools

import jax
import jax.numpy as jnp
from jax import lax
from jax.experimental import pallas as pl
from jax.experimental.pallas import tpu as pltpu
from jax.experimental.pallas import tpu_sc as plsc

N = 10000
D_IN = 128
E = 320000
NUM_GRAPHS = 64

NPAD = 10240
DUMMY = 10008
L = 128
NW = 32
NB = 80
EPAD = NW * NB * L
RPT = NPAD // 16
BS = 1280
GRID = NPAD // BS


def _sc_mesh():
    return plsc.VectorSubcoreMesh(core_axis_name="c", subcore_axis_name="s")


def _zero_vmem(buf, rows, cols):
    z = jnp.zeros((16,), jnp.float32)

    def body(i, _):
        for c in range(cols // 16):
            buf[i, pl.ds(c * 16, 16)] = z
        return 0

    lax.fori_loop(0, rows, body, 0)


def _sc_segsum(src2d, dst2d, h_pad):
    d = 64

    @functools.partial(
        pl.kernel,
        out_type=jax.ShapeDtypeStruct((NPAD, 2, d), jnp.float32),
        mesh=_sc_mesh(),
        scratch_types=[
            pltpu.VMEM((NB // 2, L), jnp.int32),
            pltpu.VMEM((NB // 2, L), jnp.int32),
            [pltpu.VMEM((L, d), jnp.float32)] * 4,
            [pltpu.SemaphoreType.DMA] * 4,
            [pltpu.SemaphoreType.DMA] * 4,
            pltpu.VMEM_SHARED((NPAD, d), jnp.float32),
            pltpu.VMEM_SHARED((NPAD, d), jnp.float32),
        ],
        compiler_params=pltpu.CompilerParams(use_tc_tiling_on_sc=False),
    )
    def k(src_hbm, dst_hbm, h_hbm, out_hbm,
          srcb, dstb, rows, sems, ssems, h_sp, acc):
        cid = lax.axis_index("c")
        sid = lax.axis_index("s")
        wid = sid * 2 + cid

        _zero_vmem(rows[0], L, d)
        base = sid * RPT
        for c in range(RPT // L):
            pltpu.sync_copy(rows[0], acc.at[pl.ds(base + c * L, L)])
            pltpu.sync_copy(h_hbm.at[pl.ds(base + c * L, L)], rows[1])
            pltpu.sync_copy(rows[1], h_sp.at[pl.ds(base + c * L, L)])
        plsc.subcore_barrier()

        nbuf = 4
        nbh = NB // 2

        def wait_g(b):
            pltpu.make_async_copy(h_sp.at[srcb.at[0]], rows[b], sems[b]).wait()

        def wait_s(b):
            pltpu.make_async_copy(rows[b], acc.at[dstb.at[0]], ssems[b]).wait()

        def body(i, _):
            for k_ in range(nbuf):
                j = i * nbuf + k_
                bn = (k_ + 2) % nbuf

                @pl.when(j >= 2)
                def _():
                    wait_s(bn)

                @pl.when(j + 2 < nbh)
                def _():
                    pltpu.async_copy(
                        h_sp.at[srcb.at[j + 2]], rows[bn], sems[bn])

                wait_g(k_)
                pltpu.async_copy(rows[k_], acc.at[dstb.at[j]],
                                 ssems[k_], add=True)
            return 0

        for half in range(2):
            pltpu.sync_copy(
                src_hbm.at[pl.ds(wid * NB + half * nbh, nbh)], srcb)
            pltpu.sync_copy(
                dst_hbm.at[pl.ds(wid * NB + half * nbh, nbh)], dstb)
            for b in range(2):
                pltpu.async_copy(h_sp.at[srcb.at[b]], rows[b], sems[b])
            lax.fori_loop(0, nbh // nbuf, body, 0)
            wait_s((nbh - 2) % nbuf)
            wait_s((nbh - 1) % nbuf)

        plsc.subcore_barrier()

        for c in range(RPT // L):
            pltpu.sync_copy(acc.at[pl.ds(base + c * L, L)], rows[c % 4])
            pltpu.sync_copy(rows[c % 4],
                            out_hbm.at[pl.ds(base + c * L, L), cid])

    return k(src2d, dst2d, h_pad)


def _tc_stats(x_pad):
    def body(x_ref, o_ref):
        i = pl.program_id(0)

        @pl.when(i == 0)
        def _():
            o_ref[...] = jnp.zeros_like(o_ref)

        xb = x_ref[...]
        s = jnp.sum(xb, axis=0, keepdims=True)
        s2 = jnp.sum(xb * xb, axis=0, keepdims=True)
        o_ref[...] += jnp.concatenate([s, s2], axis=0)

    return pl.pallas_call(
        body,
        grid=(GRID,),
        in_specs=[pl.BlockSpec((BS, D_IN), lambda i: (i, 0))],
        out_specs=pl.BlockSpec((2, D_IN), lambda i: (0, 0)),
        out_shape=jax.ShapeDtypeStruct((2, D_IN), jnp.float32),
    )(x_pad)


def _dis_block(d_ref):
    deg = d_ref[:, 0:1] + d_ref[:, 64:65]
    return lax.rsqrt(deg + 1.0)


def _tc_layer1(x_pad, stats, gamma2, beta2, w1):
    def body(x_ref, st_ref, g_ref, b_ref, w_ref, o_ref):
        xb = x_ref[...]
        mean = st_ref[0:1, :] * (1.0 / N)
        ex2 = st_ref[1:2, :] * (1.0 / N)
        inv = lax.rsqrt(ex2 - mean * mean + 1e-5)
        hb = (xb - mean) * (inv * g_ref[...]) + b_ref[...]
        o_ref[...] = jnp.dot(hb, w_ref[...],
                             preferred_element_type=jnp.float32)

    return pl.pallas_call(
        body,
        grid=(GRID,),
        in_specs=[
            pl.BlockSpec((BS, D_IN), lambda i: (i, 0)),
            pl.BlockSpec((2, D_IN), lambda i: (0, 0)),
            pl.BlockSpec((1, D_IN), lambda i: (0, 0)),
            pl.BlockSpec((1, D_IN), lambda i: (0, 0)),
            pl.BlockSpec((D_IN, 64), lambda i: (0, 0)),
        ],
        out_specs=pl.BlockSpec((BS, 64), lambda i: (i, 0)),
        out_shape=jax.ShapeDtypeStruct((NPAD, 64), jnp.float32),
    )(x_pad, stats, gamma2, beta2, w1)


def _tc_scale(g1, degs):
    def body(g_ref, d_ref, o_ref):
        o_ref[...] = g_ref[...] * _dis_block(d_ref)

    return pl.pallas_call(
        body,
        grid=(GRID,),
        in_specs=[
            pl.BlockSpec((BS, 64), lambda i: (i, 0)),
            pl.BlockSpec((BS, 128), lambda i: (i, 0)),
        ],
        out_specs=pl.BlockSpec((BS, 64), lambda i: (i, 0)),
        out_shape=jax.ShapeDtypeStruct((NPAD, 64), jnp.float32),
    )(g1, degs)


def _tc_layer(s_sum, hprev, degs, bias2, w, dout):
    din = hprev.shape[1]

    def body(s_ref, h_ref, d_ref, b_ref, w_ref, o_ref):
        dis = _dis_block(d_ref)
        sb = s_ref[...]
        t = (sb[:, :64] + sb[:, 64:] + h_ref[...]) * dis + b_ref[...]
        t = jnp.maximum(t, 0.0)
        o_ref[...] = jnp.dot(t, w_ref[...],
                             preferred_element_type=jnp.float32) * dis

    return pl.pallas_call(
        body,
        grid=(GRID,),
        in_specs=[
            pl.BlockSpec((BS, 2 * din), lambda i: (i, 0)),
            pl.BlockSpec((BS, din), lambda i: (i, 0)),
            pl.BlockSpec((BS, 128), lambda i: (i, 0)),
            pl.BlockSpec((1, din), lambda i: (0, 0)),
            pl.BlockSpec((din, dout), lambda i: (0, 0)),
        ],
        out_specs=pl.BlockSpec((BS, dout), lambda i: (i, 0)),
        out_shape=jax.ShapeDtypeStruct((NPAD, dout), jnp.float32),
    )(s_sum, hprev, degs, bias2, w)


def _tc_final(s_parts, h3, degp, b32, batch3d, wc1, bc1r, wc2, bc2r):
    def body(s_ref, h_ref, d_ref, b_ref, bt_ref, w1_ref, b1_ref,
             w2_ref, b2_ref, o_ref, pooled, cnt):
        i = pl.program_id(0)

        @pl.when(i == 0)
        def _():
            pooled[...] = jnp.zeros_like(pooled)
            cnt[...] = jnp.zeros_like(cnt)

        dis = _dis_block(d_ref)
        sb = s_ref[...]
        hb = (sb[:, :64] + sb[:, 64:] + h_ref[...]) * dis + b_ref[...]
        hb = jnp.maximum(hb, 0.0)[:, :32]
        bt = bt_ref[0]
        ohT = (lax.broadcasted_iota(jnp.int32, (NUM_GRAPHS, BS), 0)
               == bt).astype(jnp.float32)
        pooled[...] += jnp.dot(ohT, hb, preferred_element_type=jnp.float32)
        cnt[...] += jnp.sum(ohT, axis=1, keepdims=True)

        @pl.when(i == GRID - 1)
        def _():
            pm = pooled[...] / jnp.maximum(cnt[:, 0:1], 1.0)
            z1 = jnp.maximum(
                jnp.dot(pm, w1_ref[...], preferred_element_type=jnp.float32)
                + b1_ref[...], 0.0)
            o_ref[...] = jnp.dot(
                z1, w2_ref[...], preferred_element_type=jnp.float32) + b2_ref[...]

    return pl.pallas_call(
        body,
        grid=(GRID,),
        in_specs=[
            pl.BlockSpec((BS, 128), lambda i: (i, 0)),
            pl.BlockSpec((BS, 64), lambda i: (i, 0)),
            pl.BlockSpec((BS, 128), lambda i: (i, 0)),
            pl.BlockSpec((1, 64), lambda i: (0, 0)),
            pl.BlockSpec((1, 1, BS), lambda i: (i, 0, 0)),
            pl.BlockSpec((32, 16), lambda i: (0, 0)),
            pl.BlockSpec((1, 16), lambda i: (0, 0)),
            pl.BlockSpec((16, 2), lambda i: (0, 0)),
            pl.BlockSpec((1, 2), lambda i: (0, 0)),
        ],
        out_specs=pl.BlockSpec((NUM_GRAPHS, 2), lambda i: (0, 0)),
        out_shape=jax.ShapeDtypeStruct((NUM_GRAPHS, 2), jnp.float32),
        scratch_shapes=[
            pltpu.VMEM((NUM_GRAPHS, 32), jnp.float32),
            pltpu.VMEM((NUM_GRAPHS, 8), jnp.float32),
        ],
    )(s_parts, h3, degp, b32, batch3d, wc1, bc1r, wc2, bc2r)


def kernel(x, edge_index, batch, gamma, beta,
           W1, b1, W2, b2, W3, b3, Wc1, bc1, Wc2, bc2):
    x_pad = jnp.pad(x, ((0, NPAD - N), (0, 0)))
    e_pad = jnp.pad(edge_index, ((0, 0), (0, EPAD - E)),
                    constant_values=DUMMY)
    src2d = e_pad[0].reshape(EPAD // L, L)
    dst2d = e_pad[1].reshape(EPAD // L, L)
    batch3d = jnp.pad(batch, (0, NPAD - N),
                      constant_values=NUM_GRAPHS).reshape(GRID, 1, BS)

    ones = jnp.ones((NPAD, 64), jnp.float32)
    degp = _sc_segsum(src2d, dst2d, ones)
    stats = _tc_stats(x_pad)
    g1 = _tc_layer1(x_pad, stats, gamma.reshape(1, D_IN),
                    beta.reshape(1, D_IN), W1)
    degs = degp.reshape(NPAD, 128)
    h1 = _tc_scale(g1, degs)
    s1 = _sc_segsum(src2d, dst2d, h1).reshape(NPAD, 128)
    h2 = _tc_layer(s1, h1, degs, b1.reshape(1, 64), W2, 64)
    s2 = _sc_segsum(src2d, dst2d, h2).reshape(NPAD, 128)
    w3p = jnp.pad(W3, ((0, 0), (0, 64 - W3.shape[1])))
    h3 = _tc_layer(s2, h2, degs, b2.reshape(1, 64), w3p, 64)
    s3 = _sc_segsum(src2d, dst2d, h3).reshape(NPAD, 128)
    b3p = jnp.pad(b3, (0, 64 - b3.shape[0])).reshape(1, 64)
    return _tc_final(s3, h3, degs, b3p, batch3d,
                     Wc1, bc1.reshape(1, 16), Wc2, bc2.reshape(1, 2))


# speedup vs baseline: 1.2781x; 1.2781x over previous
import functools

import jax
import jax.numpy as jnp
from jax import lax
from jax.experimental import pallas as pl
from jax.experimental.pallas import tpu as pltpu
from jax.experimental.pallas import tpu_sc as plsc

N = 10000
D_IN = 128
E = 320000
NUM_GRAPHS = 64

NPAD = 10240
DUMMY = 10008
L = 128
NW = 32
NB = 80
EPAD = NW * NB * L
RPT = NPAD // 16
BS = 1280
GRID = NPAD // BS


def _sc_mesh():
    return plsc.VectorSubcoreMesh(core_axis_name="c", subcore_axis_name="s")


def _zero_vmem(buf, rows, cols):
    z = jnp.zeros((16,), jnp.float32)

    def body(i, _):
        for c in range(cols // 16):
            buf[i, pl.ds(c * 16, 16)] = z
        return 0

    lax.fori_loop(0, rows, body, 0)


def _sc_segsum(src2d, dst2d, h_pad):
    d = 64

    @functools.partial(
        pl.kernel,
        out_type=jax.ShapeDtypeStruct((2, NPAD, d), jnp.float32),
        mesh=_sc_mesh(),
        scratch_types=[
            pltpu.VMEM((NB // 2, L), jnp.int32),
            pltpu.VMEM((NB // 2, L), jnp.int32),
            [pltpu.VMEM((L, d), jnp.float32)] * 4,
            [pltpu.SemaphoreType.DMA] * 4,
            [pltpu.SemaphoreType.DMA] * 4,
            pltpu.VMEM_SHARED((NPAD, d), jnp.float32),
            pltpu.VMEM_SHARED((NPAD, d), jnp.float32),
        ],
        compiler_params=pltpu.CompilerParams(use_tc_tiling_on_sc=False),
    )
    def k(src_hbm, dst_hbm, h_hbm, out_hbm,
          srcb, dstb, rows, sems, ssems, h_sp, acc):
        cid = lax.axis_index("c")
        sid = lax.axis_index("s")
        wid = sid * 2 + cid

        _zero_vmem(rows[0], L, d)
        base = sid * RPT
        for c in range(RPT // L):
            pltpu.sync_copy(rows[0], acc.at[pl.ds(base + c * L, L)])
            pltpu.sync_copy(h_hbm.at[pl.ds(base + c * L, L)], rows[1])
            pltpu.sync_copy(rows[1], h_sp.at[pl.ds(base + c * L, L)])
        plsc.subcore_barrier()

        nbuf = 4
        nbh = NB // 2

        def wait_g(b):
            pltpu.make_async_copy(h_sp.at[srcb.at[0]], rows[b], sems[b]).wait()

        def wait_s(b):
            pltpu.make_async_copy(rows[b], acc.at[dstb.at[0]], ssems[b]).wait()

        def body(i, _):
            for k_ in range(nbuf):
                j = i * nbuf + k_
                bn = (k_ + 2) % nbuf

                @pl.when(j >= 2)
                def _():
                    wait_s(bn)

                @pl.when(j + 2 < nbh)
                def _():
                    pltpu.async_copy(
                        h_sp.at[srcb.at[j + 2]], rows[bn], sems[bn])

                wait_g(k_)
                pltpu.async_copy(rows[k_], acc.at[dstb.at[j]],
                                 ssems[k_], add=True)
            return 0

        for half in range(2):
            pltpu.sync_copy(
                src_hbm.at[pl.ds(wid * NB + half * nbh, nbh)], srcb)
            pltpu.sync_copy(
                dst_hbm.at[pl.ds(wid * NB + half * nbh, nbh)], dstb)
            for b in range(2):
                pltpu.async_copy(h_sp.at[srcb.at[b]], rows[b], sems[b])
            lax.fori_loop(0, nbh // nbuf, body, 0)
            wait_s((nbh - 2) % nbuf)
            wait_s((nbh - 1) % nbuf)

        plsc.subcore_barrier()

        for c in range(RPT // L):
            pltpu.sync_copy(acc.at[pl.ds(base + c * L, L)], rows[c % 4])
            pltpu.sync_copy(rows[c % 4],
                            out_hbm.at[cid, pl.ds(base + c * L, L)])

    return k(src2d, dst2d, h_pad)


def _tc_stats(x_pad):
    def body(x_ref, o_ref):
        i = pl.program_id(0)

        @pl.when(i == 0)
        def _():
            o_ref[...] = jnp.zeros_like(o_ref)

        xb = x_ref[...]
        s = jnp.sum(xb, axis=0, keepdims=True)
        s2 = jnp.sum(xb * xb, axis=0, keepdims=True)
        o_ref[...] += jnp.concatenate([s, s2], axis=0)

    return pl.pallas_call(
        body,
        grid=(GRID,),
        in_specs=[pl.BlockSpec((BS, D_IN), lambda i: (i, 0))],
        out_specs=pl.BlockSpec((2, D_IN), lambda i: (0, 0)),
        out_shape=jax.ShapeDtypeStruct((2, D_IN), jnp.float32),
    )(x_pad)


def _dis_block(d_ref):
    deg = d_ref[0] + d_ref[1]
    return lax.rsqrt(deg[:, 0:1] + 1.0)


def _tc_layer1(x_pad, stats, gamma2, beta2, w1):
    def body(x_ref, st_ref, g_ref, b_ref, w_ref, o_ref):
        xb = x_ref[...]
        mean = st_ref[0:1, :] * (1.0 / N)
        ex2 = st_ref[1:2, :] * (1.0 / N)
        inv = lax.rsqrt(ex2 - mean * mean + 1e-5)
        hb = (xb - mean) * (inv * g_ref[...]) + b_ref[...]
        o_ref[...] = jnp.dot(hb, w_ref[...],
                             preferred_element_type=jnp.float32)

    return pl.pallas_call(
        body,
        grid=(GRID,),
        in_specs=[
            pl.BlockSpec((BS, D_IN), lambda i: (i, 0)),
            pl.BlockSpec((2, D_IN), lambda i: (0, 0)),
            pl.BlockSpec((1, D_IN), lambda i: (0, 0)),
            pl.BlockSpec((1, D_IN), lambda i: (0, 0)),
            pl.BlockSpec((D_IN, 64), lambda i: (0, 0)),
        ],
        out_specs=pl.BlockSpec((BS, 64), lambda i: (i, 0)),
        out_shape=jax.ShapeDtypeStruct((NPAD, 64), jnp.float32),
    )(x_pad, stats, gamma2, beta2, w1)


def _tc_scale(g1, degs):
    def body(g_ref, d_ref, o_ref):
        o_ref[...] = g_ref[...] * _dis_block(d_ref)

    return pl.pallas_call(
        body,
        grid=(GRID,),
        in_specs=[
            pl.BlockSpec((BS, 64), lambda i: (i, 0)),
            pl.BlockSpec((2, BS, 64), lambda i: (0, i, 0)),
        ],
        out_specs=pl.BlockSpec((BS, 64), lambda i: (i, 0)),
        out_shape=jax.ShapeDtypeStruct((NPAD, 64), jnp.float32),
    )(g1, degs)


def _tc_layer(s_sum, hprev, degs, bias2, w, dout):
    din = hprev.shape[1]

    def body(s_ref, h_ref, d_ref, b_ref, w_ref, o_ref):
        dis = _dis_block(d_ref)
        t = (s_ref[0] + s_ref[1] + h_ref[...]) * dis + b_ref[...]
        t = jnp.maximum(t, 0.0)
        o_ref[...] = jnp.dot(t, w_ref[...],
                             preferred_element_type=jnp.float32) * dis

    return pl.pallas_call(
        body,
        grid=(GRID,),
        in_specs=[
            pl.BlockSpec((2, BS, din), lambda i: (0, i, 0)),
            pl.BlockSpec((BS, din), lambda i: (i, 0)),
            pl.BlockSpec((2, BS, 64), lambda i: (0, i, 0)),
            pl.BlockSpec((1, din), lambda i: (0, 0)),
            pl.BlockSpec((din, dout), lambda i: (0, 0)),
        ],
        out_specs=pl.BlockSpec((BS, dout), lambda i: (i, 0)),
        out_shape=jax.ShapeDtypeStruct((NPAD, dout), jnp.float32),
    )(s_sum, hprev, degs, bias2, w)


def _tc_final(s_parts, h3, degp, b32, batch3d, wc1, bc1r, wc2, bc2r):
    def body(s_ref, h_ref, d_ref, b_ref, bt_ref, w1_ref, b1_ref,
             w2_ref, b2_ref, o_ref, pooled, cnt):
        i = pl.program_id(0)

        @pl.when(i == 0)
        def _():
            pooled[...] = jnp.zeros_like(pooled)
            cnt[...] = jnp.zeros_like(cnt)

        dis = _dis_block(d_ref)
        hb = (s_ref[0] + s_ref[1] + h_ref[...]) * dis + b_ref[...]
        hb = jnp.maximum(hb, 0.0)[:, :32]
        bt = bt_ref[0]
        ohT = (lax.broadcasted_iota(jnp.int32, (NUM_GRAPHS, BS), 0)
               == bt).astype(jnp.float32)
        pooled[...] += jnp.dot(ohT, hb, preferred_element_type=jnp.float32)
        cnt[...] += jnp.sum(ohT, axis=1, keepdims=True)

        @pl.when(i == GRID - 1)
        def _():
            pm = pooled[...] / jnp.maximum(cnt[:, 0:1], 1.0)
            z1 = jnp.maximum(
                jnp.dot(pm, w1_ref[...], preferred_element_type=jnp.float32)
                + b1_ref[...], 0.0)
            o_ref[...] = jnp.dot(
                z1, w2_ref[...], preferred_element_type=jnp.float32) + b2_ref[...]

    return pl.pallas_call(
        body,
        grid=(GRID,),
        in_specs=[
            pl.BlockSpec((2, BS, 64), lambda i: (0, i, 0)),
            pl.BlockSpec((BS, 64), lambda i: (i, 0)),
            pl.BlockSpec((2, BS, 64), lambda i: (0, i, 0)),
            pl.BlockSpec((1, 64), lambda i: (0, 0)),
            pl.BlockSpec((1, 1, BS), lambda i: (i, 0, 0)),
            pl.BlockSpec((32, 16), lambda i: (0, 0)),
            pl.BlockSpec((1, 16), lambda i: (0, 0)),
            pl.BlockSpec((16, 2), lambda i: (0, 0)),
            pl.BlockSpec((1, 2), lambda i: (0, 0)),
        ],
        out_specs=pl.BlockSpec((NUM_GRAPHS, 2), lambda i: (0, 0)),
        out_shape=jax.ShapeDtypeStruct((NUM_GRAPHS, 2), jnp.float32),
        scratch_shapes=[
            pltpu.VMEM((NUM_GRAPHS, 32), jnp.float32),
            pltpu.VMEM((NUM_GRAPHS, 8), jnp.float32),
        ],
    )(s_parts, h3, degp, b32, batch3d, wc1, bc1r, wc2, bc2r)


def kernel(x, edge_index, batch, gamma, beta,
           W1, b1, W2, b2, W3, b3, Wc1, bc1, Wc2, bc2):
    x_pad = jnp.pad(x, ((0, NPAD - N), (0, 0)))
    e_pad = jnp.pad(edge_index, ((0, 0), (0, EPAD - E)),
                    constant_values=DUMMY)
    src2d = e_pad[0].reshape(EPAD // L, L)
    dst2d = e_pad[1].reshape(EPAD // L, L)
    batch3d = jnp.pad(batch, (0, NPAD - N),
                      constant_values=NUM_GRAPHS).reshape(GRID, 1, BS)

    ones = jnp.ones((NPAD, 64), jnp.float32)
    degp = _sc_segsum(src2d, dst2d, ones)
    stats = _tc_stats(x_pad)
    g1 = _tc_layer1(x_pad, stats, gamma.reshape(1, D_IN),
                    beta.reshape(1, D_IN), W1)
    degs = degp
    h1 = _tc_scale(g1, degs)
    s1 = _sc_segsum(src2d, dst2d, h1)
    h2 = _tc_layer(s1, h1, degs, b1.reshape(1, 64), W2, 64)
    s2 = _sc_segsum(src2d, dst2d, h2)
    w3p = jnp.pad(W3, ((0, 0), (0, 64 - W3.shape[1])))
    h3 = _tc_layer(s2, h2, degs, b2.reshape(1, 64), w3p, 64)
    s3 = _sc_segsum(src2d, dst2d, h3)
    b3p = jnp.pad(b3, (0, 64 - b3.shape[0])).reshape(1, 64)
    return _tc_final(s3, h3, degs, b3p, batch3d,
                     Wc1, bc1.reshape(1, 16), Wc2, bc2.reshape(1, 2))


# speedup vs baseline: 1.3624x; 1.0659x over previous
import functools

import jax
import jax.numpy as jnp
from jax import lax
from jax.experimental import pallas as pl
from jax.experimental.pallas import tpu as pltpu
from jax.experimental.pallas import tpu_sc as plsc

N = 10000
D_IN = 128
E = 320000
NUM_GRAPHS = 64

NPAD = 10240
DUMMY = 10008
L = 128
NW = 32
NB = 80
EPAD = NW * NB * L
RPT = NPAD // 16
BS = 1280
GRID = NPAD // BS


def _sc_mesh():
    return plsc.VectorSubcoreMesh(core_axis_name="c", subcore_axis_name="s")


def _zero_vmem(buf, rows, cols):
    z = jnp.zeros((16,), jnp.float32)

    def body(i, _):
        for c in range(cols // 16):
            buf[i, pl.ds(c * 16, 16)] = z
        return 0

    lax.fori_loop(0, rows, body, 0)


def _sc_segsum(src2d, dst2d, h_pad):
    d = 64

    @functools.partial(
        pl.kernel,
        out_type=jax.ShapeDtypeStruct((2, NPAD, d), jnp.float32),
        mesh=_sc_mesh(),
        scratch_types=[
            pltpu.VMEM((NB // 2, L), jnp.int32),
            pltpu.VMEM((NB // 2, L), jnp.int32),
            [pltpu.VMEM((L, d), jnp.float32)] * 4,
            [pltpu.SemaphoreType.DMA] * 4,
            [pltpu.SemaphoreType.DMA] * 4,
            pltpu.VMEM_SHARED((NPAD, d), jnp.float32),
            pltpu.VMEM_SHARED((NPAD, d), jnp.float32),
        ],
        compiler_params=pltpu.CompilerParams(use_tc_tiling_on_sc=False),
    )
    def k(src_hbm, dst_hbm, h_hbm, out_hbm,
          srcb, dstb, rows, sems, ssems, h_sp, acc):
        cid = lax.axis_index("c")
        sid = lax.axis_index("s")
        wid = sid * 2 + cid

        nbh = NB // 2
        base = sid * RPT
        nch = RPT // L

        pltpu.async_copy(src_hbm.at[pl.ds(wid * NB, nbh)], srcb, ssems[0])
        pltpu.async_copy(dst_hbm.at[pl.ds(wid * NB, nbh)], dstb, ssems[1])

        _zero_vmem(rows[0], L, d)
        for c in range(3):
            pltpu.async_copy(h_hbm.at[pl.ds(base + c * L, L)],
                             rows[1 + c], sems[1 + c])
        for c in range(nch):
            pltpu.sync_copy(rows[0], acc.at[pl.ds(base + c * L, L)])
            b = 1 + (c % 3)
            pltpu.make_async_copy(h_hbm.at[pl.ds(base, L)],
                                  rows[b], sems[b]).wait()
            pltpu.sync_copy(rows[b], h_sp.at[pl.ds(base + c * L, L)])
            if c + 3 < nch:
                pltpu.async_copy(h_hbm.at[pl.ds(base + (c + 3) * L, L)],
                                 rows[b], sems[b])
        plsc.subcore_barrier()
        pltpu.make_async_copy(src_hbm.at[pl.ds(0, nbh)], srcb,
                              ssems[0]).wait()
        pltpu.make_async_copy(dst_hbm.at[pl.ds(0, nbh)], dstb,
                              ssems[1]).wait()

        nbuf = 4

        def wait_g(b):
            pltpu.make_async_copy(h_sp.at[srcb.at[0]], rows[b], sems[b]).wait()

        def wait_s(b):
            pltpu.make_async_copy(rows[b], acc.at[dstb.at[0]], ssems[b]).wait()

        def body(i, _):
            for k_ in range(nbuf):
                j = i * nbuf + k_
                bn = (k_ + 2) % nbuf

                @pl.when(j >= 2)
                def _():
                    wait_s(bn)

                @pl.when(j + 2 < nbh)
                def _():
                    pltpu.async_copy(
                        h_sp.at[srcb.at[j + 2]], rows[bn], sems[bn])

                wait_g(k_)
                pltpu.async_copy(rows[k_], acc.at[dstb.at[j]],
                                 ssems[k_], add=True)
            return 0

        for half in range(2):
            if half == 1:
                pltpu.sync_copy(
                    src_hbm.at[pl.ds(wid * NB + nbh, nbh)], srcb)
                pltpu.sync_copy(
                    dst_hbm.at[pl.ds(wid * NB + nbh, nbh)], dstb)
            for b in range(2):
                pltpu.async_copy(h_sp.at[srcb.at[b]], rows[b], sems[b])
            lax.fori_loop(0, nbh // nbuf, body, 0)
            wait_s((nbh - 2) % nbuf)
            wait_s((nbh - 1) % nbuf)

        plsc.subcore_barrier()

        pltpu.async_copy(acc.at[pl.ds(base, L)], rows[0], sems[0])
        for c in range(nch):
            pltpu.make_async_copy(acc.at[pl.ds(base, L)],
                                  rows[c % 4], sems[c % 4]).wait()
            if c + 1 < nch:
                pltpu.async_copy(acc.at[pl.ds(base + (c + 1) * L, L)],
                                 rows[(c + 1) % 4], sems[(c + 1) % 4])
            pltpu.sync_copy(rows[c % 4],
                            out_hbm.at[cid, pl.ds(base + c * L, L)])

    return k(src2d, dst2d, h_pad)


def _tc_stats(x_pad):
    def body(x_ref, o_ref):
        i = pl.program_id(0)

        @pl.when(i == 0)
        def _():
            o_ref[...] = jnp.zeros_like(o_ref)

        xb = x_ref[...]
        s = jnp.sum(xb, axis=0, keepdims=True)
        s2 = jnp.sum(xb * xb, axis=0, keepdims=True)
        o_ref[...] += jnp.concatenate([s, s2], axis=0)

    return pl.pallas_call(
        body,
        grid=(GRID,),
        in_specs=[pl.BlockSpec((BS, D_IN), lambda i: (i, 0))],
        out_specs=pl.BlockSpec((2, D_IN), lambda i: (0, 0)),
        out_shape=jax.ShapeDtypeStruct((2, D_IN), jnp.float32),
    )(x_pad)


def _dis_block(d_ref):
    deg = d_ref[0] + d_ref[1]
    return lax.rsqrt(deg[:, 0:1] + 1.0)


def _tc_layer1(x_pad, stats, gamma2, beta2, w1):
    def body(x_ref, st_ref, g_ref, b_ref, w_ref, o_ref):
        xb = x_ref[...]
        mean = st_ref[0:1, :] * (1.0 / N)
        ex2 = st_ref[1:2, :] * (1.0 / N)
        inv = lax.rsqrt(ex2 - mean * mean + 1e-5)
        hb = (xb - mean) * (inv * g_ref[...]) + b_ref[...]
        o_ref[...] = jnp.dot(hb, w_ref[...],
                             preferred_element_type=jnp.float32)

    return pl.pallas_call(
        body,
        grid=(GRID,),
        in_specs=[
            pl.BlockSpec((BS, D_IN), lambda i: (i, 0)),
            pl.BlockSpec((2, D_IN), lambda i: (0, 0)),
            pl.BlockSpec((1, D_IN), lambda i: (0, 0)),
            pl.BlockSpec((1, D_IN), lambda i: (0, 0)),
            pl.BlockSpec((D_IN, 64), lambda i: (0, 0)),
        ],
        out_specs=pl.BlockSpec((BS, 64), lambda i: (i, 0)),
        out_shape=jax.ShapeDtypeStruct((NPAD, 64), jnp.float32),
    )(x_pad, stats, gamma2, beta2, w1)


def _tc_scale(g1, degs):
    def body(g_ref, d_ref, o_ref):
        o_ref[...] = g_ref[...] * _dis_block(d_ref)

    return pl.pallas_call(
        body,
        grid=(GRID,),
        in_specs=[
            pl.BlockSpec((BS, 64), lambda i: (i, 0)),
            pl.BlockSpec((2, BS, 64), lambda i: (0, i, 0)),
        ],
        out_specs=pl.BlockSpec((BS, 64), lambda i: (i, 0)),
        out_shape=jax.ShapeDtypeStruct((NPAD, 64), jnp.float32),
    )(g1, degs)


def _tc_layer(s_sum, hprev, degs, bias2, w, dout):
    din = hprev.shape[1]

    def body(s_ref, h_ref, d_ref, b_ref, w_ref, o_ref):
        dis = _dis_block(d_ref)
        t = (s_ref[0] + s_ref[1] + h_ref[...]) * dis + b_ref[...]
        t = jnp.maximum(t, 0.0)
        o_ref[...] = jnp.dot(t, w_ref[...],
                             preferred_element_type=jnp.float32) * dis

    return pl.pallas_call(
        body,
        grid=(GRID,),
        in_specs=[
            pl.BlockSpec((2, BS, din), lambda i: (0, i, 0)),
            pl.BlockSpec((BS, din), lambda i: (i, 0)),
            pl.BlockSpec((2, BS, 64), lambda i: (0, i, 0)),
            pl.BlockSpec((1, din), lambda i: (0, 0)),
            pl.BlockSpec((din, dout), lambda i: (0, 0)),
        ],
        out_specs=pl.BlockSpec((BS, dout), lambda i: (i, 0)),
        out_shape=jax.ShapeDtypeStruct((NPAD, dout), jnp.float32),
    )(s_sum, hprev, degs, bias2, w)


def _tc_final(s_parts, h3, degp, b32, batch3d, wc1, bc1r, wc2, bc2r):
    def body(s_ref, h_ref, d_ref, b_ref, bt_ref, w1_ref, b1_ref,
             w2_ref, b2_ref, o_ref, pooled, cnt):
        i = pl.program_id(0)

        @pl.when(i == 0)
        def _():
            pooled[...] = jnp.zeros_like(pooled)
            cnt[...] = jnp.zeros_like(cnt)

        dis = _dis_block(d_ref)
        hb = (s_ref[0] + s_ref[1] + h_ref[...]) * dis + b_ref[...]
        hb = jnp.maximum(hb, 0.0)[:, :32]
        bt = bt_ref[0]
        ohT = (lax.broadcasted_iota(jnp.int32, (NUM_GRAPHS, BS), 0)
               == bt).astype(jnp.float32)
        pooled[...] += jnp.dot(ohT, hb, preferred_element_type=jnp.float32)
        cnt[...] += jnp.sum(ohT, axis=1, keepdims=True)

        @pl.when(i == GRID - 1)
        def _():
            pm = pooled[...] / jnp.maximum(cnt[:, 0:1], 1.0)
            z1 = jnp.maximum(
                jnp.dot(pm, w1_ref[...], preferred_element_type=jnp.float32)
                + b1_ref[...], 0.0)
            o_ref[...] = jnp.dot(
                z1, w2_ref[...], preferred_element_type=jnp.float32) + b2_ref[...]

    return pl.pallas_call(
        body,
        grid=(GRID,),
        in_specs=[
            pl.BlockSpec((2, BS, 64), lambda i: (0, i, 0)),
            pl.BlockSpec((BS, 64), lambda i: (i, 0)),
            pl.BlockSpec((2, BS, 64), lambda i: (0, i, 0)),
            pl.BlockSpec((1, 64), lambda i: (0, 0)),
            pl.BlockSpec((1, 1, BS), lambda i: (i, 0, 0)),
            pl.BlockSpec((32, 16), lambda i: (0, 0)),
            pl.BlockSpec((1, 16), lambda i: (0, 0)),
            pl.BlockSpec((16, 2), lambda i: (0, 0)),
            pl.BlockSpec((1, 2), lambda i: (0, 0)),
        ],
        out_specs=pl.BlockSpec((NUM_GRAPHS, 2), lambda i: (0, 0)),
        out_shape=jax.ShapeDtypeStruct((NUM_GRAPHS, 2), jnp.float32),
        scratch_shapes=[
            pltpu.VMEM((NUM_GRAPHS, 32), jnp.float32),
            pltpu.VMEM((NUM_GRAPHS, 8), jnp.float32),
        ],
    )(s_parts, h3, degp, b32, batch3d, wc1, bc1r, wc2, bc2r)


def kernel(x, edge_index, batch, gamma, beta,
           W1, b1, W2, b2, W3, b3, Wc1, bc1, Wc2, bc2):
    x_pad = jnp.pad(x, ((0, NPAD - N), (0, 0)))
    e_pad = jnp.pad(edge_index, ((0, 0), (0, EPAD - E)),
                    constant_values=DUMMY)
    src2d = e_pad[0].reshape(EPAD // L, L)
    dst2d = e_pad[1].reshape(EPAD // L, L)
    batch3d = jnp.pad(batch, (0, NPAD - N),
                      constant_values=NUM_GRAPHS).reshape(GRID, 1, BS)

    ones = jnp.ones((NPAD, 64), jnp.float32)
    degp = _sc_segsum(src2d, dst2d, ones)
    stats = _tc_stats(x_pad)
    g1 = _tc_layer1(x_pad, stats, gamma.reshape(1, D_IN),
                    beta.reshape(1, D_IN), W1)
    degs = degp
    h1 = _tc_scale(g1, degs)
    s1 = _sc_segsum(src2d, dst2d, h1)
    h2 = _tc_layer(s1, h1, degs, b1.reshape(1, 64), W2, 64)
    s2 = _sc_segsum(src2d, dst2d, h2)
    w3p = jnp.pad(W3, ((0, 0), (0, 64 - W3.shape[1])))
    h3 = _tc_layer(s2, h2, degs, b2.reshape(1, 64), w3p, 64)
    s3 = _sc_segsum(src2d, dst2d, h3)
    b3p = jnp.pad(b3, (0, 64 - b3.shape[0])).reshape(1, 64)
    return _tc_final(s3, h3, degs, b3p, batch3d,
                     Wc1, bc1.reshape(1, 16), Wc2, bc2.reshape(1, 2))


# speedup vs baseline: 1.3712x; 1.0065x over previous
import functools

import jax
import jax.numpy as jnp
from jax import lax
from jax.experimental import pallas as pl
from jax.experimental.pallas import tpu as pltpu
from jax.experimental.pallas import tpu_sc as plsc

N = 10000
D_IN = 128
E = 320000
NUM_GRAPHS = 64

NPAD = 10240
DUMMY = 10008
L = 128
NW = 32
NB = 80
EPAD = NW * NB * L
RPT = NPAD // 16
BS = 1280
GRID = NPAD // BS


def _sc_mesh():
    return plsc.VectorSubcoreMesh(core_axis_name="c", subcore_axis_name="s")


def _zero_vmem(buf, rows, cols):
    z = jnp.zeros((16,), jnp.float32)

    def body(i, _):
        for c in range(cols // 16):
            buf[i, pl.ds(c * 16, 16)] = z
        return 0

    lax.fori_loop(0, rows, body, 0)


def _sc_segsum(src2d, dst2d, h_pad):
    d = 64

    @functools.partial(
        pl.kernel,
        out_type=jax.ShapeDtypeStruct((2, NPAD, d), jnp.float32),
        mesh=_sc_mesh(),
        scratch_types=[
            pltpu.VMEM((NB // 2, L), jnp.int32),
            pltpu.VMEM((NB // 2, L), jnp.int32),
            [pltpu.VMEM((L, d), jnp.float32)] * 4,
            [pltpu.SemaphoreType.DMA] * 4,
            [pltpu.SemaphoreType.DMA] * 4,
            pltpu.VMEM_SHARED((NPAD, d), jnp.float32),
            pltpu.VMEM_SHARED((NPAD, d), jnp.float32),
        ],
        compiler_params=pltpu.CompilerParams(use_tc_tiling_on_sc=False),
    )
    def k(src_hbm, dst_hbm, h_hbm, out_hbm,
          srcb, dstb, rows, sems, ssems, h_sp, acc):
        cid = lax.axis_index("c")
        sid = lax.axis_index("s")
        wid = sid * 2 + cid

        nbh = NB // 2
        base = sid * RPT
        nch = RPT // L

        pltpu.async_copy(src_hbm.at[pl.ds(wid * NB, nbh)], srcb, ssems[0])
        pltpu.async_copy(dst_hbm.at[pl.ds(wid * NB, nbh)], dstb, ssems[1])

        _zero_vmem(rows[0], L, d)
        for c in range(3):
            pltpu.async_copy(h_hbm.at[pl.ds(base + c * L, L)],
                             rows[1 + c], sems[1 + c])
        for c in range(nch):
            pltpu.sync_copy(rows[0], acc.at[pl.ds(base + c * L, L)])
            b = 1 + (c % 3)
            pltpu.make_async_copy(h_hbm.at[pl.ds(base, L)],
                                  rows[b], sems[b]).wait()
            pltpu.sync_copy(rows[b], h_sp.at[pl.ds(base + c * L, L)])
            if c + 3 < nch:
                pltpu.async_copy(h_hbm.at[pl.ds(base + (c + 3) * L, L)],
                                 rows[b], sems[b])
        plsc.subcore_barrier()
        pltpu.make_async_copy(src_hbm.at[pl.ds(0, nbh)], srcb,
                              ssems[0]).wait()
        pltpu.make_async_copy(dst_hbm.at[pl.ds(0, nbh)], dstb,
                              ssems[1]).wait()

        nbuf = 4

        def wait_g(b):
            pltpu.make_async_copy(h_sp.at[srcb.at[0]], rows[b], sems[b]).wait()

        def wait_s(b):
            pltpu.make_async_copy(rows[b], acc.at[dstb.at[0]], ssems[b]).wait()

        def body(i, _):
            for k_ in range(nbuf):
                j = i * nbuf + k_
                bn = (k_ + 2) % nbuf

                @pl.when(j >= 2)
                def _():
                    wait_s(bn)

                @pl.when(j + 2 < nbh)
                def _():
                    pltpu.async_copy(
                        h_sp.at[srcb.at[j + 2]], rows[bn], sems[bn])

                wait_g(k_)
                pltpu.async_copy(rows[k_], acc.at[dstb.at[j]],
                                 ssems[k_], add=True)
            return 0

        for half in range(2):
            if half == 1:
                pltpu.sync_copy(
                    src_hbm.at[pl.ds(wid * NB + nbh, nbh)], srcb)
                pltpu.sync_copy(
                    dst_hbm.at[pl.ds(wid * NB + nbh, nbh)], dstb)
            for b in range(2):
                pltpu.async_copy(h_sp.at[srcb.at[b]], rows[b], sems[b])
            lax.fori_loop(0, nbh // nbuf, body, 0)
            wait_s((nbh - 2) % nbuf)
            wait_s((nbh - 1) % nbuf)

        plsc.subcore_barrier()

        pltpu.async_copy(acc.at[pl.ds(base, L)], rows[0], sems[0])
        for c in range(nch):
            pltpu.make_async_copy(acc.at[pl.ds(base, L)],
                                  rows[c % 4], sems[c % 4]).wait()
            if c + 1 < nch:
                pltpu.async_copy(acc.at[pl.ds(base + (c + 1) * L, L)],
                                 rows[(c + 1) % 4], sems[(c + 1) % 4])
            pltpu.sync_copy(rows[c % 4],
                            out_hbm.at[cid, pl.ds(base + c * L, L)])

    return k(src2d, dst2d, h_pad)


def _tc_stats(x_pad):
    def body(x_ref, o_ref):
        i = pl.program_id(0)

        @pl.when(i == 0)
        def _():
            o_ref[...] = jnp.zeros_like(o_ref)

        xb = x_ref[...]
        s = jnp.sum(xb, axis=0, keepdims=True)
        s2 = jnp.sum(xb * xb, axis=0, keepdims=True)
        o_ref[...] += jnp.concatenate([s, s2], axis=0)

    return pl.pallas_call(
        body,
        grid=(GRID,),
        in_specs=[pl.BlockSpec((BS, D_IN), lambda i: (i, 0))],
        out_specs=pl.BlockSpec((2, D_IN), lambda i: (0, 0)),
        out_shape=jax.ShapeDtypeStruct((2, D_IN), jnp.float32),
    )(x_pad)


def _dis_block(d_ref):
    return lax.rsqrt(d_ref[:, 0:1] + 1.0)


def _tc_layer1(x_pad, stats, gamma2, beta2, w1):
    def body(x_ref, st_ref, g_ref, b_ref, w_ref, o_ref):
        xb = x_ref[...]
        mean = st_ref[0:1, :] * (1.0 / N)
        ex2 = st_ref[1:2, :] * (1.0 / N)
        inv = lax.rsqrt(ex2 - mean * mean + 1e-5)
        hb = (xb - mean) * (inv * g_ref[...]) + b_ref[...]
        o_ref[...] = jnp.dot(hb, w_ref[...],
                             preferred_element_type=jnp.float32)

    return pl.pallas_call(
        body,
        grid=(GRID,),
        in_specs=[
            pl.BlockSpec((BS, D_IN), lambda i: (i, 0)),
            pl.BlockSpec((2, D_IN), lambda i: (0, 0)),
            pl.BlockSpec((1, D_IN), lambda i: (0, 0)),
            pl.BlockSpec((1, D_IN), lambda i: (0, 0)),
            pl.BlockSpec((D_IN, 64), lambda i: (0, 0)),
        ],
        out_specs=pl.BlockSpec((BS, 64), lambda i: (i, 0)),
        out_shape=jax.ShapeDtypeStruct((NPAD, 64), jnp.float32),
    )(x_pad, stats, gamma2, beta2, w1)


def _tc_scale(g1, degp):
    def body(g_ref, d_ref, o_ref, dc_ref):
        deg = d_ref[0] + d_ref[1]
        degc = deg[:, 0:1]
        o_ref[...] = g_ref[...] * lax.rsqrt(degc + 1.0)
        dc_ref[...] = degc * jnp.ones((1, 8), jnp.float32)

    return pl.pallas_call(
        body,
        grid=(GRID,),
        in_specs=[
            pl.BlockSpec((BS, 64), lambda i: (i, 0)),
            pl.BlockSpec((2, BS, 64), lambda i: (0, i, 0)),
        ],
        out_specs=[
            pl.BlockSpec((BS, 64), lambda i: (i, 0)),
            pl.BlockSpec((BS, 8), lambda i: (i, 0)),
        ],
        out_shape=[
            jax.ShapeDtypeStruct((NPAD, 64), jnp.float32),
            jax.ShapeDtypeStruct((NPAD, 8), jnp.float32),
        ],
    )(g1, degp)


def _tc_layer(s_sum, hprev, degs, bias2, w, dout):
    din = hprev.shape[1]

    def body(s_ref, h_ref, d_ref, b_ref, w_ref, o_ref):
        dis = _dis_block(d_ref)
        t = (s_ref[0] + s_ref[1] + h_ref[...]) * dis + b_ref[...]
        t = jnp.maximum(t, 0.0)
        o_ref[...] = jnp.dot(t, w_ref[...],
                             preferred_element_type=jnp.float32) * dis

    return pl.pallas_call(
        body,
        grid=(GRID,),
        in_specs=[
            pl.BlockSpec((2, BS, din), lambda i: (0, i, 0)),
            pl.BlockSpec((BS, din), lambda i: (i, 0)),
            pl.BlockSpec((BS, 8), lambda i: (i, 0)),
            pl.BlockSpec((1, din), lambda i: (0, 0)),
            pl.BlockSpec((din, dout), lambda i: (0, 0)),
        ],
        out_specs=pl.BlockSpec((BS, dout), lambda i: (i, 0)),
        out_shape=jax.ShapeDtypeStruct((NPAD, dout), jnp.float32),
    )(s_sum, hprev, degs, bias2, w)


def _tc_final(s_parts, h3, degp, b32, batch3d, wc1, bc1r, wc2, bc2r):
    def body(s_ref, h_ref, d_ref, b_ref, bt_ref, w1_ref, b1_ref,
             w2_ref, b2_ref, o_ref, pooled, cnt):
        i = pl.program_id(0)

        @pl.when(i == 0)
        def _():
            pooled[...] = jnp.zeros_like(pooled)
            cnt[...] = jnp.zeros_like(cnt)

        dis = _dis_block(d_ref)
        hb = (s_ref[0] + s_ref[1] + h_ref[...]) * dis + b_ref[...]
        hb = jnp.maximum(hb, 0.0)[:, :32]
        bt = bt_ref[0]
        ohT = (lax.broadcasted_iota(jnp.int32, (NUM_GRAPHS, BS), 0)
               == bt).astype(jnp.float32)
        pooled[...] += jnp.dot(ohT, hb, preferred_element_type=jnp.float32)
        cnt[...] += jnp.sum(ohT, axis=1, keepdims=True)

        @pl.when(i == GRID - 1)
        def _():
            pm = pooled[...] / jnp.maximum(cnt[:, 0:1], 1.0)
            z1 = jnp.maximum(
                jnp.dot(pm, w1_ref[...], preferred_element_type=jnp.float32)
                + b1_ref[...], 0.0)
            o_ref[...] = jnp.dot(
                z1, w2_ref[...], preferred_element_type=jnp.float32) + b2_ref[...]

    return pl.pallas_call(
        body,
        grid=(GRID,),
        in_specs=[
            pl.BlockSpec((2, BS, 64), lambda i: (0, i, 0)),
            pl.BlockSpec((BS, 64), lambda i: (i, 0)),
            pl.BlockSpec((BS, 8), lambda i: (i, 0)),
            pl.BlockSpec((1, 64), lambda i: (0, 0)),
            pl.BlockSpec((1, 1, BS), lambda i: (i, 0, 0)),
            pl.BlockSpec((32, 16), lambda i: (0, 0)),
            pl.BlockSpec((1, 16), lambda i: (0, 0)),
            pl.BlockSpec((16, 2), lambda i: (0, 0)),
            pl.BlockSpec((1, 2), lambda i: (0, 0)),
        ],
        out_specs=pl.BlockSpec((NUM_GRAPHS, 2), lambda i: (0, 0)),
        out_shape=jax.ShapeDtypeStruct((NUM_GRAPHS, 2), jnp.float32),
        scratch_shapes=[
            pltpu.VMEM((NUM_GRAPHS, 32), jnp.float32),
            pltpu.VMEM((NUM_GRAPHS, 8), jnp.float32),
        ],
    )(s_parts, h3, degp, b32, batch3d, wc1, bc1r, wc2, bc2r)


def kernel(x, edge_index, batch, gamma, beta,
           W1, b1, W2, b2, W3, b3, Wc1, bc1, Wc2, bc2):
    x_pad = jnp.pad(x, ((0, NPAD - N), (0, 0)))
    e_pad = jnp.pad(edge_index, ((0, 0), (0, EPAD - E)),
                    constant_values=DUMMY)
    src2d = e_pad[0].reshape(EPAD // L, L)
    dst2d = e_pad[1].reshape(EPAD // L, L)
    batch3d = jnp.pad(batch, (0, NPAD - N),
                      constant_values=NUM_GRAPHS).reshape(GRID, 1, BS)

    ones = jnp.ones((NPAD, 64), jnp.float32)
    degp = _sc_segsum(src2d, dst2d, ones)
    stats = _tc_stats(x_pad)
    g1 = _tc_layer1(x_pad, stats, gamma.reshape(1, D_IN),
                    beta.reshape(1, D_IN), W1)
    h1, degs = _tc_scale(g1, degp)
    s1 = _sc_segsum(src2d, dst2d, h1)
    h2 = _tc_layer(s1, h1, degs, b1.reshape(1, 64), W2, 64)
    s2 = _sc_segsum(src2d, dst2d, h2)
    w3p = jnp.pad(W3, ((0, 0), (0, 64 - W3.shape[1])))
    h3 = _tc_layer(s2, h2, degs, b2.reshape(1, 64), w3p, 64)
    s3 = _sc_segsum(src2d, dst2d, h3)
    b3p = jnp.pad(b3, (0, 64 - b3.shape[0])).reshape(1, 64)
    return _tc_final(s3, h3, degs, b3p, batch3d,
                     Wc1, bc1.reshape(1, 16), Wc2, bc2.reshape(1, 2))
